# baseline (device time: 77086 ns/iter reference)
import jax
import jax.numpy as jnp
from jax import lax
from jax.experimental import pallas as pl
from jax.experimental.pallas import tpu as pltpu

NZ = 4
NP = 4
B, S, H, Dh, Dr = 2, 512, 16, 128, 32
D = 2048
DCS = 512 // NZ
BS = B * S
HL = H // NP
HB = HL * Dh
NBO = 4
BD = D // NBO
SCALE = (Dh + Dr) ** -0.5
BF16 = jnp.bfloat16
F32 = jnp.float32


def _body(x_ref, wdkv_ref, wuk_ref, wuv_ref, wq_ref, wqr_ref, wkr_ref, wo_ref,
          out_ref,
          x_scr, c_gath, w_cast, w_gath, q_tile, q_own, wqr_tile, qr_own,
          kr_buf, o_own, o_gath, wtile,
          z_send_sems, z_recv_sems, o_send_sems, o_recv_sems, local_sems):
    my_x = lax.axis_index("x")
    my_y = lax.axis_index("y")
    my_z = lax.axis_index("z")
    my_p = my_x * 2 + my_y

    cp_x = pltpu.make_async_copy(x_ref, x_scr, local_sems.at[3])
    cp_x.start()
    cp_q = pltpu.make_async_copy(
        wq_ref.at[:, pl.ds(my_p * HB, HB)], q_tile, local_sems.at[1])
    cp_q.start()
    cp_qr = pltpu.make_async_copy(
        wqr_ref.at[:, pl.ds(my_p * HL * Dr, HL * Dr)], wqr_tile,
        local_sems.at[2])
    cp_qr.start()

    barrier = pltpu.get_barrier_semaphore()
    for d in range(1, NZ):
        pl.semaphore_signal(
            barrier, inc=1,
            device_id=(my_x, my_y, lax.rem(my_z + d, NZ)),
            device_id_type=pl.DeviceIdType.MESH,
        )
    for d in range(1, NP):
        pt = lax.rem(my_p + d, NP)
        pl.semaphore_signal(
            barrier, inc=1,
            device_id=(pt // 2, lax.rem(pt, 2), my_z),
            device_id_type=pl.DeviceIdType.MESH,
        )
    pl.semaphore_wait(barrier, NZ - 1 + NP - 1)

    cp_x.wait()
    xb = x_scr[...].reshape(BS, D).astype(BF16)

    for h in range(H):
        g = h * 2 * Dh
        w_cast[:, g:g + Dh] = wuk_ref[:, h * Dh:(h + 1) * Dh].astype(BF16)
        w_cast[:, g + Dh:g + 2 * Dh] = (
            wuv_ref[:, h * Dh:(h + 1) * Dh].astype(BF16))
    cp_w = pltpu.make_async_copy(
        w_cast.at[:, pl.ds(my_p * 2 * HB, 2 * HB)], w_gath.at[0],
        local_sems.at[0])
    cp_w.start()
    c_gath[0, :, :] = jnp.dot(
        xb, wdkv_ref[...].astype(BF16), preferred_element_type=F32
    ).astype(BF16)
    cp_w.wait()

    z_rdmas = []
    for d in range(1, NZ):
        tz = lax.rem(my_z + d, NZ)
        for j, buf in enumerate((w_gath, c_gath)):
            i = (d - 1) * 2 + j
            r = pltpu.make_async_remote_copy(
                src_ref=buf.at[0],
                dst_ref=buf.at[d],
                send_sem=z_send_sems.at[i],
                recv_sem=z_recv_sems.at[i],
                device_id=(my_x, my_y, tz),
                device_id_type=pl.DeviceIdType.MESH,
            )
            r.start()
            z_rdmas.append(r)

    kr_buf[...] = jnp.dot(xb, wkr_ref[...].astype(BF16),
                          preferred_element_type=F32).astype(BF16)
    cp_qr.wait()
    qr_own[...] = jnp.dot(xb, wqr_tile[...].astype(BF16),
                          preferred_element_type=F32).astype(BF16)
    cp_q.wait()
    q_own[...] = jnp.dot(xb, q_tile[...].astype(BF16),
                         preferred_element_type=F32).astype(BF16)

    for r in z_rdmas:
        r.wait_recv()

    row_offs = [my_p] + [lax.rem(my_p + NP - i, NP) for i in range(1, NP)]
    wo_copies = [
        pltpu.make_async_copy(
            wo_ref.at[pl.ds(row_offs[i] * HB, HB), :],
            wtile.at[i % 2],
            local_sems.at[i % 2],
        )
        for i in range(NP)
    ]
    wo_copies[0].start()
    wo_copies[1].start()

    o_rdmas = []
    for b in range(B):
        r0 = b * S
        kr = kr_buf[r0:r0 + S, :]
        for hl in range(HL):
            c0 = hl * Dh
            kv = jnp.zeros((S, 2 * Dh), F32)
            for d in range(NZ):
                kv += jnp.dot(c_gath[d, r0:r0 + S, :],
                              w_gath[d, :, hl * 2 * Dh:(hl + 1) * 2 * Dh],
                              preferred_element_type=F32)
            k_bh = kv[:, :Dh]
            v_bh = kv[:, Dh:]
            q = q_own[r0:r0 + S, c0:c0 + Dh]
            qr = qr_own[r0:r0 + S, hl * Dr:(hl + 1) * Dr]
            s = lax.dot_general(q, k_bh.astype(BF16),
                                (((1,), (1,)), ((), ())),
                                preferred_element_type=F32)
            s += lax.dot_general(qr, kr, (((1,), (1,)), ((), ())),
                                 preferred_element_type=F32)
            s *= SCALE
            m = jnp.max(s, axis=1, keepdims=True)
            p = jnp.exp(s - m)
            denom = jnp.sum(p, axis=1, keepdims=True)
            o = lax.dot_general(p.astype(BF16), v_bh.astype(BF16),
                                (((1,), (0,)), ((), ())),
                                preferred_element_type=F32)
            o = o / denom
            o_own[r0:r0 + S, c0:c0 + Dh] = o.astype(BF16)
        for d in range(1, NP):
            pt = lax.rem(my_p + d, NP)
            i = (d - 1) * B + b
            r = pltpu.make_async_remote_copy(
                src_ref=o_own.at[pl.ds(r0, S), :],
                dst_ref=o_gath.at[d, pl.ds(r0, S), :],
                send_sem=o_send_sems.at[i],
                recv_sem=o_recv_sems.at[i],
                device_id=(pt // 2, lax.rem(pt, 2), my_z),
                device_id_type=pl.DeviceIdType.MESH,
            )
            r.start()
            o_rdmas.append(r)

    for i in range(NP):
        if 2 <= i + 1 < NP:
            wo_copies[i + 1].start()
        if i > 0:
            for b in range(B):
                o_rdmas[b * (NP - 1) + (i - 1)].wait_recv()
        wo_copies[i].wait()
        lhs = o_own[...] if i == 0 else o_gath[i, :, :]
        wt = wtile[i % 2, :, :].astype(BF16)
        for jc in range(2):
            cs = jc * (D // 2)
            contrib = jnp.dot(
                lhs, wt[:, cs:cs + D // 2],
                preferred_element_type=F32).reshape(B, S, D // 2)
            if i == 0:
                out_ref[:, :, cs:cs + D // 2] = contrib
            else:
                out_ref[:, :, cs:cs + D // 2] = (
                    out_ref[:, :, cs:cs + D // 2] + contrib)

    for r in z_rdmas:
        r.wait_send()
    for r in o_rdmas:
        r.wait_send()


def kernel(x, Wdkv, Wuk, Wuv, Wq, Wqr, Wkr, Wo):
    vmem = pl.BlockSpec(memory_space=pltpu.MemorySpace.VMEM)
    hbm = pl.BlockSpec(memory_space=pltpu.MemorySpace.HBM)
    return pl.pallas_call(
        _body,
        out_shape=jax.ShapeDtypeStruct((B, S, D), F32),
        in_specs=[hbm, vmem, vmem, vmem, hbm, hbm, vmem, hbm],
        out_specs=vmem,
        scratch_shapes=[
            pltpu.VMEM((B, S, D), F32),
            pltpu.VMEM((NZ, BS, DCS), BF16),
            pltpu.VMEM((DCS, 2 * D), BF16),
            pltpu.VMEM((NZ, DCS, 2 * HB), BF16),
            pltpu.VMEM((D, HB), F32),
            pltpu.VMEM((BS, HB), BF16),
            pltpu.VMEM((D, HL * Dr), F32),
            pltpu.VMEM((BS, HL * Dr), BF16),
            pltpu.VMEM((BS, Dr), BF16),
            pltpu.VMEM((BS, HB), BF16),
            pltpu.VMEM((NP, BS, HB), BF16),
            pltpu.VMEM((2, HB, D), F32),
            pltpu.SemaphoreType.DMA((2 * (NZ - 1),)),
            pltpu.SemaphoreType.DMA((2 * (NZ - 1),)),
            pltpu.SemaphoreType.DMA(((NP - 1) * B,)),
            pltpu.SemaphoreType.DMA(((NP - 1) * B,)),
            pltpu.SemaphoreType.DMA((4,)),
        ],
        compiler_params=pltpu.CompilerParams(
            collective_id=0,
            vmem_limit_bytes=63 * 1024 * 1024,
        ),
    )(x, Wdkv, Wuk, Wuv, Wq, Wqr, Wkr, Wo)


# device time: 73974 ns/iter; 1.0421x vs baseline; 1.0421x over previous
import jax
import jax.numpy as jnp
from jax import lax
from jax.experimental import pallas as pl
from jax.experimental.pallas import tpu as pltpu

NZ = 4
NP = 4
B, S, H, Dh, Dr = 2, 512, 16, 128, 32
D = 2048
DCS = 512 // NZ
BS = B * S
HL = H // NP
HB = HL * Dh
NBO = 4
BD = D // NBO
SCALE = (Dh + Dr) ** -0.5
BF16 = jnp.bfloat16
F32 = jnp.float32


def _body(x_ref, wdkv_ref, wuk_ref, wuv_ref, wq_ref, wqr_ref, wkr_ref, wo_ref,
          out_ref,
          x_scr, c_gath, w_cast, w_gath, q_tile, q_own, wqr_tile, qr_own,
          kr_buf, o_own, o_gath, wtile,
          z_send_sems, z_recv_sems, o_send_sems, o_recv_sems, local_sems):
    my_x = lax.axis_index("x")
    my_y = lax.axis_index("y")
    my_z = lax.axis_index("z")
    my_p = my_x * 2 + my_y

    cp_x = pltpu.make_async_copy(x_ref, x_scr, local_sems.at[3])
    cp_x.start()
    cp_q = pltpu.make_async_copy(
        wq_ref.at[:, pl.ds(my_p * HB, HB)], q_tile, local_sems.at[1])
    cp_q.start()
    cp_qr = pltpu.make_async_copy(
        wqr_ref.at[:, pl.ds(my_p * HL * Dr, HL * Dr)], wqr_tile,
        local_sems.at[2])
    cp_qr.start()

    barrier = pltpu.get_barrier_semaphore()
    for d in range(1, NZ):
        pl.semaphore_signal(
            barrier, inc=1,
            device_id=(my_x, my_y, lax.rem(my_z + d, NZ)),
            device_id_type=pl.DeviceIdType.MESH,
        )
    for d in range(1, NP):
        pt = lax.rem(my_p + d, NP)
        pl.semaphore_signal(
            barrier, inc=1,
            device_id=(pt // 2, lax.rem(pt, 2), my_z),
            device_id_type=pl.DeviceIdType.MESH,
        )
    pl.semaphore_wait(barrier, NZ - 1 + NP - 1)

    for h in range(H):
        g = h * 2 * Dh
        w_cast[:, g:g + Dh] = wuk_ref[:, h * Dh:(h + 1) * Dh].astype(BF16)
        w_cast[:, g + Dh:g + 2 * Dh] = (
            wuv_ref[:, h * Dh:(h + 1) * Dh].astype(BF16))
    cp_w = pltpu.make_async_copy(
        w_cast.at[:, pl.ds(my_p * 2 * HB, 2 * HB)], w_gath.at[0],
        local_sems.at[0])
    cp_w.start()
    cp_w.wait()

    def z_rdma(buf, i, d, tz):
        return pltpu.make_async_remote_copy(
            src_ref=buf.at[0],
            dst_ref=buf.at[d],
            send_sem=z_send_sems.at[i],
            recv_sem=z_recv_sems.at[i],
            device_id=(my_x, my_y, tz),
            device_id_type=pl.DeviceIdType.MESH,
        )

    z_rdmas = []
    for d in range(1, NZ):
        r = z_rdma(w_gath, d - 1, d, lax.rem(my_z + d, NZ))
        r.start()
        z_rdmas.append(r)

    cp_x.wait()
    xb = x_scr[...].reshape(BS, D).astype(BF16)
    c_gath[0, :, :] = jnp.dot(
        xb, wdkv_ref[...].astype(BF16), preferred_element_type=F32
    ).astype(BF16)
    for d in range(1, NZ):
        r = z_rdma(c_gath, (NZ - 1) + d - 1, d, lax.rem(my_z + d, NZ))
        r.start()
        z_rdmas.append(r)

    kr_buf[...] = jnp.dot(xb, wkr_ref[...].astype(BF16),
                          preferred_element_type=F32).astype(BF16)
    cp_qr.wait()
    qr_own[...] = jnp.dot(xb, wqr_tile[...].astype(BF16),
                          preferred_element_type=F32).astype(BF16)
    cp_q.wait()
    q_own[...] = jnp.dot(xb, q_tile[...].astype(BF16),
                         preferred_element_type=F32).astype(BF16)

    for r in z_rdmas:
        r.wait_recv()

    row_offs = [my_p] + [lax.rem(my_p + NP - i, NP) for i in range(1, NP)]
    wo_copies = [
        pltpu.make_async_copy(
            wo_ref.at[pl.ds(row_offs[i] * HB, HB), :],
            wtile.at[i % 2],
            local_sems.at[i % 2],
        )
        for i in range(NP)
    ]
    wo_copies[0].start()
    wo_copies[1].start()

    o_rdmas = {d: [] for d in range(1, NP)}
    for b in range(B):
        r0 = b * S
        kr = kr_buf[r0:r0 + S, :]
        for hl in range(HL):
            c0 = hl * Dh
            kv = jnp.zeros((S, 2 * Dh), F32)
            for d in range(NZ):
                kv += jnp.dot(c_gath[d, r0:r0 + S, :],
                              w_gath[d, :, hl * 2 * Dh:(hl + 1) * 2 * Dh],
                              preferred_element_type=F32)
            k_bh = kv[:, :Dh]
            v_bh = kv[:, Dh:]
            q = q_own[r0:r0 + S, c0:c0 + Dh]
            qr = qr_own[r0:r0 + S, hl * Dr:(hl + 1) * Dr]
            s = lax.dot_general(q, k_bh.astype(BF16),
                                (((1,), (1,)), ((), ())),
                                preferred_element_type=F32)
            s += lax.dot_general(qr, kr, (((1,), (1,)), ((), ())),
                                 preferred_element_type=F32)
            s *= SCALE
            m = jnp.max(s, axis=1, keepdims=True)
            p = jnp.exp(s - m)
            denom = jnp.sum(p, axis=1, keepdims=True)
            o = lax.dot_general(p.astype(BF16), v_bh.astype(BF16),
                                (((1,), (0,)), ((), ())),
                                preferred_element_type=F32)
            o = o / denom
            o_own[r0:r0 + S, c0:c0 + Dh] = o.astype(BF16)
            for d in range(1, NP):
                pt = lax.rem(my_p + d, NP)
                i = (d - 1) * B * HL + b * HL + hl
                r = pltpu.make_async_remote_copy(
                    src_ref=o_own.at[pl.ds(r0, S), pl.ds(c0, Dh)],
                    dst_ref=o_gath.at[d, pl.ds(r0, S), pl.ds(c0, Dh)],
                    send_sem=o_send_sems.at[i],
                    recv_sem=o_recv_sems.at[i],
                    device_id=(pt // 2, lax.rem(pt, 2), my_z),
                    device_id_type=pl.DeviceIdType.MESH,
                )
                r.start()
                o_rdmas[d].append(r)

    for i in range(NP):
        if 2 <= i + 1 < NP:
            wo_copies[i + 1].start()
        if i > 0:
            for r in o_rdmas[i]:
                r.wait_recv()
        wo_copies[i].wait()
        lhs = o_own[...] if i == 0 else o_gath[i, :, :]
        wt = wtile[i % 2, :, :].astype(BF16)
        for jc in range(2):
            cs = jc * (D // 2)
            contrib = jnp.dot(
                lhs, wt[:, cs:cs + D // 2],
                preferred_element_type=F32).reshape(B, S, D // 2)
            if i == 0:
                out_ref[:, :, cs:cs + D // 2] = contrib
            else:
                out_ref[:, :, cs:cs + D // 2] = (
                    out_ref[:, :, cs:cs + D // 2] + contrib)

    for r in z_rdmas:
        r.wait_send()
    for d in range(1, NP):
        for r in o_rdmas[d]:
            r.wait_send()


def kernel(x, Wdkv, Wuk, Wuv, Wq, Wqr, Wkr, Wo):
    vmem = pl.BlockSpec(memory_space=pltpu.MemorySpace.VMEM)
    hbm = pl.BlockSpec(memory_space=pltpu.MemorySpace.HBM)
    return pl.pallas_call(
        _body,
        out_shape=jax.ShapeDtypeStruct((B, S, D), F32),
        in_specs=[hbm, vmem, vmem, vmem, hbm, hbm, vmem, hbm],
        out_specs=vmem,
        scratch_shapes=[
            pltpu.VMEM((B, S, D), F32),
            pltpu.VMEM((NZ, BS, DCS), BF16),
            pltpu.VMEM((DCS, 2 * D), BF16),
            pltpu.VMEM((NZ, DCS, 2 * HB), BF16),
            pltpu.VMEM((D, HB), F32),
            pltpu.VMEM((BS, HB), BF16),
            pltpu.VMEM((D, HL * Dr), F32),
            pltpu.VMEM((BS, HL * Dr), BF16),
            pltpu.VMEM((BS, Dr), BF16),
            pltpu.VMEM((BS, HB), BF16),
            pltpu.VMEM((NP, BS, HB), BF16),
            pltpu.VMEM((2, HB, D), F32),
            pltpu.SemaphoreType.DMA((2 * (NZ - 1),)),
            pltpu.SemaphoreType.DMA((2 * (NZ - 1),)),
            pltpu.SemaphoreType.DMA(((NP - 1) * B * HL,)),
            pltpu.SemaphoreType.DMA(((NP - 1) * B * HL,)),
            pltpu.SemaphoreType.DMA((4,)),
        ],
        compiler_params=pltpu.CompilerParams(
            collective_id=0,
            vmem_limit_bytes=63 * 1024 * 1024,
        ),
    )(x, Wdkv, Wuk, Wuv, Wq, Wqr, Wkr, Wo)
